# Initial kernel scaffold; baseline (speedup 1.0000x reference)
import jax
import jax.numpy as jnp
from jax.experimental import pallas as pl


def kernel(x, edge_index, W0, b0, W1, b1, W2, b2):
    raise NotImplementedError("write your pallas kernel here")


# speedup vs baseline: 7.2248x; 7.2248x over previous
import functools

import jax
import jax.numpy as jnp
from jax import lax
from jax.experimental import pallas as pl
from jax.experimental.pallas import tpu as pltpu
from jax.experimental.pallas import tpu_sc as plsc

N_NODES = 10000
D = 128
E = 320000
LANES = 16

N_PAD = 10240
EPT_ROWS = 80
N_TILES = 32
E_PAD = N_TILES * EPT_ROWS * 128
IDX_ROWS = E_PAD // 128
ROWS_PER_TILE = N_PAD // 16

TC_BLOCK = 1024
TC_GRID = N_PAD // TC_BLOCK

_mesh = plsc.VectorSubcoreMesh(core_axis_name="c", subcore_axis_name="s")


@functools.partial(
    pl.kernel,
    mesh=_mesh,
    out_type=jax.ShapeDtypeStruct((2, N_PAD, D), jnp.float32),
    scratch_types=[
        pltpu.VMEM((EPT_ROWS, 128), jnp.int32),
        pltpu.VMEM((128, D), jnp.float32),
        pltpu.VMEM_SHARED((N_PAD, D), jnp.float32),
    ],
)
def _deg_kernel(dst_hbm, out_hbm, dst_v, ones_v, sdeg):
    cid = lax.axis_index("c")
    sid = lax.axis_index("s")
    tid = cid * 16 + sid

    one = jnp.full((LANES,), 1.0, jnp.float32)
    zero = jnp.zeros((LANES,), jnp.float32)

    def zfill(i, carry):
        for k in range(D // LANES):
            ones_v[i, pl.ds(k * LANES, LANES)] = zero
        return carry

    lax.fori_loop(0, 128, zfill, 0)

    for k in range(ROWS_PER_TILE // 128):
        pltpu.sync_copy(
            ones_v, sdeg.at[pl.ds(sid * ROWS_PER_TILE + k * 128, 128)])
    pltpu.sync_copy(dst_hbm.at[pl.ds(tid * EPT_ROWS, EPT_ROWS)], dst_v)

    def ofill(i, carry):
        for k in range(D // LANES):
            ones_v[i, pl.ds(k * LANES, LANES)] = one
        return carry

    lax.fori_loop(0, 128, ofill, 0)

    plsc.subcore_barrier()

    def body(j, carry):
        pltpu.sync_copy(ones_v, sdeg.at[dst_v.at[j]], add=True)
        return carry

    lax.fori_loop(0, EPT_ROWS, body, 0)

    plsc.subcore_barrier()
    pltpu.sync_copy(
        sdeg.at[pl.ds(sid * ROWS_PER_TILE, ROWS_PER_TILE)],
        out_hbm.at[cid, pl.ds(sid * ROWS_PER_TILE, ROWS_PER_TILE)],
    )


@functools.partial(
    pl.kernel,
    mesh=_mesh,
    out_type=jax.ShapeDtypeStruct((2, N_PAD, D), jnp.float32),
    scratch_types=[
        pltpu.VMEM((EPT_ROWS, 128), jnp.int32),
        pltpu.VMEM((EPT_ROWS, 128), jnp.int32),
        pltpu.VMEM((128, D), jnp.float32),
        pltpu.VMEM_SHARED((N_PAD, D), jnp.float32),
        pltpu.SemaphoreType.DMA,
    ],
)
def _agg_kernel(g_hbm, src_hbm, dst_hbm, out_hbm,
                src_v, dst_v, rows_v, acc, sem):
    cid = lax.axis_index("c")
    sid = lax.axis_index("s")
    tid = cid * 16 + sid

    zero = jnp.zeros((LANES,), jnp.float32)

    def fill(i, carry):
        for k in range(D // LANES):
            rows_v[i, pl.ds(k * LANES, LANES)] = zero
        return carry

    lax.fori_loop(0, 128, fill, 0)

    for k in range(ROWS_PER_TILE // 128):
        pltpu.sync_copy(
            rows_v, acc.at[pl.ds(sid * ROWS_PER_TILE + k * 128, 128)])
    pltpu.sync_copy(src_hbm.at[pl.ds(tid * EPT_ROWS, EPT_ROWS)], src_v)
    pltpu.sync_copy(dst_hbm.at[pl.ds(tid * EPT_ROWS, EPT_ROWS)], dst_v)

    plsc.subcore_barrier()

    def body(j, carry):
        pltpu.async_copy(g_hbm.at[src_v.at[j]], rows_v, sem).wait()
        pltpu.sync_copy(rows_v, acc.at[dst_v.at[j]], add=True)
        return carry

    lax.fori_loop(0, EPT_ROWS, body, 0)

    plsc.subcore_barrier()
    pltpu.sync_copy(
        acc.at[pl.ds(sid * ROWS_PER_TILE, ROWS_PER_TILE)],
        out_hbm.at[cid, pl.ds(sid * ROWS_PER_TILE, ROWS_PER_TILE)],
    )


def _first_body(x_ref, w_ref, degp_ref, g_ref, dis_ref):
    deg = degp_ref[0] + degp_ref[1] + 1.0
    dis = lax.rsqrt(deg)
    h = jnp.dot(x_ref[...], w_ref[...], preferred_element_type=jnp.float32)
    g_ref[...] = h * dis
    dis_ref[...] = dis


def _mid_body(accp_ref, g_ref, dis_ref, b_ref, w_ref, o_ref):
    s = accp_ref[0] + accp_ref[1] + g_ref[...]
    dis = dis_ref[...]
    a = jnp.maximum(s * dis + b_ref[...], 0.0)
    o_ref[...] = jnp.dot(
        a, w_ref[...], preferred_element_type=jnp.float32) * dis


def _last_body(accp_ref, g_ref, dis_ref, b_ref, z_ref):
    s = accp_ref[0] + accp_ref[1] + g_ref[...]
    z_ref[...] = s * dis_ref[...] + b_ref[...]


_first_tc = pl.pallas_call(
    _first_body,
    grid=(TC_GRID,),
    in_specs=[
        pl.BlockSpec((TC_BLOCK, D), lambda i: (i, 0)),
        pl.BlockSpec((D, D), lambda i: (0, 0)),
        pl.BlockSpec((2, TC_BLOCK, D), lambda i: (0, i, 0)),
    ],
    out_specs=[
        pl.BlockSpec((TC_BLOCK, D), lambda i: (i, 0)),
        pl.BlockSpec((TC_BLOCK, D), lambda i: (i, 0)),
    ],
    out_shape=[
        jax.ShapeDtypeStruct((N_PAD, D), jnp.float32),
        jax.ShapeDtypeStruct((N_PAD, D), jnp.float32),
    ],
)

_mid_tc = pl.pallas_call(
    _mid_body,
    grid=(TC_GRID,),
    in_specs=[
        pl.BlockSpec((2, TC_BLOCK, D), lambda i: (0, i, 0)),
        pl.BlockSpec((TC_BLOCK, D), lambda i: (i, 0)),
        pl.BlockSpec((TC_BLOCK, D), lambda i: (i, 0)),
        pl.BlockSpec((1, D), lambda i: (0, 0)),
        pl.BlockSpec((D, D), lambda i: (0, 0)),
    ],
    out_specs=pl.BlockSpec((TC_BLOCK, D), lambda i: (i, 0)),
    out_shape=jax.ShapeDtypeStruct((N_PAD, D), jnp.float32),
)

_last_tc = pl.pallas_call(
    _last_body,
    grid=(TC_GRID,),
    in_specs=[
        pl.BlockSpec((2, TC_BLOCK, D), lambda i: (0, i, 0)),
        pl.BlockSpec((TC_BLOCK, D), lambda i: (i, 0)),
        pl.BlockSpec((TC_BLOCK, D), lambda i: (i, 0)),
        pl.BlockSpec((1, D), lambda i: (0, 0)),
    ],
    out_specs=pl.BlockSpec((TC_BLOCK, D), lambda i: (i, 0)),
    out_shape=jax.ShapeDtypeStruct((N_PAD, D), jnp.float32),
)


def kernel(x, edge_index, W0, b0, W1, b1, W2, b2):
    src = edge_index[0]
    dst = edge_index[1]
    pad_e = E_PAD - E
    src_r = jnp.pad(src, (0, pad_e), constant_values=N_NODES).reshape(
        IDX_ROWS, 128)
    dst_r = jnp.pad(dst, (0, pad_e), constant_values=N_NODES).reshape(
        IDX_ROWS, 128)
    x_pad = jnp.pad(x, ((0, N_PAD - N_NODES), (0, 0)))

    degp = _deg_kernel(dst_r)
    g0, dis16 = _first_tc(x_pad, W0, degp)
    a0 = _agg_kernel(g0, src_r, dst_r)
    g1 = _mid_tc(a0, g0, dis16, b0.reshape(1, D), W1)
    a1 = _agg_kernel(g1, src_r, dst_r)
    g2 = _mid_tc(a1, g1, dis16, b1.reshape(1, D), W2)
    a2 = _agg_kernel(g2, src_r, dst_r)
    z = _last_tc(a2, g2, dis16, b2.reshape(1, D))
    return z[:N_NODES]
